# baseline (speedup 1.0000x reference)
import functools

import jax
import jax.numpy as jnp
from jax import lax
from jax.experimental import pallas as pl
from jax.experimental.pallas import tpu as pltpu
from jax.experimental.pallas import tpu_sc as plsc

NUM_EMB = 1_000_000
DIM = 16
BATCH = 16384
BAG = 50

NUM_CORES = 2
NUM_SUBCORES = 16
NW = NUM_CORES * NUM_SUBCORES
BPW = BATCH // NW
IPW = BPW * BAG
GROUP_BAGS = 4
GROUP = GROUP_BAGS * BAG
SPLIT = 128
GPW = BPW // GROUP_BAGS
NBUF = 4


@functools.partial(
    pl.kernel,
    mesh=plsc.VectorSubcoreMesh(core_axis_name="c", subcore_axis_name="s"),
    out_type=jax.ShapeDtypeStruct((BATCH, DIM), jnp.float32),
    compiler_params=pltpu.CompilerParams(use_tc_tiling_on_sc=False),
    scratch_types=[
        pltpu.VMEM((IPW,), jnp.int32),
        pltpu.VMEM((BPW, DIM), jnp.float32),
    ] + [pltpu.VMEM((GROUP, DIM), jnp.float32) for _ in range(NBUF)]
      + [pltpu.SemaphoreType.DMA for _ in range(NBUF)],
)
def _embedding_bag_sc(idx_hbm, tbl_hbm, out_hbm, idx_v, out_v, *bufs):
    rows = bufs[:NBUF]
    sems = bufs[NBUF:]
    wid = lax.axis_index("s") * NUM_CORES + lax.axis_index("c")

    pltpu.sync_copy(idx_hbm.at[pl.ds(wid * IPW, IPW)], idx_v)

    def copies(g, b):
        base = GROUP * g
        return (
            pltpu.make_async_copy(
                tbl_hbm.at[idx_v.at[pl.ds(base, SPLIT)]],
                rows[b].at[pl.ds(0, SPLIT)], sems[b]),
            pltpu.make_async_copy(
                tbl_hbm.at[idx_v.at[pl.ds(base + SPLIT, GROUP - SPLIT)]],
                rows[b].at[pl.ds(SPLIT, GROUP - SPLIT)], sems[b]),
        )

    def start(g, b):
        for c in copies(g, b):
            c.start()

    def finish(g, b):
        for c in copies(g, b):
            c.wait()
        r = rows[b]
        for j in range(GROUP_BAGS):
            parts = []
            for c in range(5):
                base = BAG * j + 10 * c
                acc = r[base]
                for k in range(base + 1, base + 10):
                    acc = acc + r[k]
                parts.append(acc)
            total = (parts[0] + parts[1]) + (parts[2] + parts[3]) + parts[4]
            out_v[GROUP_BAGS * g + j] = total * jnp.float32(1.0 / BAG)

    for b in range(NBUF):
        start(b, b)

    def body(i, carry):
        for b in range(NBUF):
            g = NBUF * i + b
            finish(g, b)
            start(g + NBUF, b)
        return carry

    lax.fori_loop(0, GPW // NBUF - 1, body, 0)

    for b in range(NBUF):
        finish(GPW - NBUF + b, b)

    pltpu.sync_copy(out_v, out_hbm.at[pl.ds(wid * BPW, BPW)])


def kernel(input, weight):
    return _embedding_bag_sc(input.astype(jnp.int32).reshape(-1), weight)
